# baseline (speedup 1.0000x reference)
import functools

import jax
import jax.numpy as jnp
from jax import lax
from jax.experimental import pallas as pl
from jax.experimental.pallas import tpu as pltpu
from jax.experimental.pallas import tpu_sc as plsc

VOCAB = 1000000
DIM = 16
B = 16384
NEG = 5

NC = 2
NS = 16
NW = NC * NS
CH = 128

BPW = B // NW
VPW = (NEG + 1) * BPW
UCH = BPW // CH
VCH = VPW // CH


def _sc_gather(u_emb, v_emb, cidx2, vxidx2):
    mesh = plsc.VectorSubcoreMesh(core_axis_name="c", subcore_axis_name="s")

    @functools.partial(
        pl.kernel,
        mesh=mesh,
        compiler_params=pltpu.CompilerParams(use_tc_tiling_on_sc=False),
        out_type=[
            jax.ShapeDtypeStruct((B, DIM), jnp.float32),
            jax.ShapeDtypeStruct(((NEG + 1) * B, DIM), jnp.float32),
        ],
        scratch_types=[
            pltpu.VMEM((UCH, CH), jnp.int32),
            pltpu.VMEM((VCH, CH), jnp.int32),
            pltpu.VMEM((BPW, DIM), jnp.float32),
            pltpu.VMEM((VPW, DIM), jnp.float32),
            pltpu.SemaphoreType.DMA,
            pltpu.SemaphoreType.DMA,
        ],
    )
    def k(u_hbm, v_hbm, ci_hbm, vx_hbm, u_out, vx_out,
          ci_v, vx_v, ur_v, vr_v, sem_u, sem_v):
        wid = lax.axis_index("s") * NC + lax.axis_index("c")
        pltpu.sync_copy(ci_hbm.at[pl.ds(wid * UCH, UCH)], ci_v)
        pltpu.sync_copy(vx_hbm.at[pl.ds(wid * VCH, VCH)], vx_v)
        descs = []
        for j in range(UCH):
            descs.append(pltpu.async_copy(
                u_hbm.at[ci_v.at[j]], ur_v.at[pl.ds(j * CH, CH)], sem_u))
        for j in range(VCH):
            descs.append(pltpu.async_copy(
                v_hbm.at[vx_v.at[j]], vr_v.at[pl.ds(j * CH, CH)], sem_v))
        for d in descs:
            d.wait()
        pltpu.sync_copy(ur_v, u_out.at[pl.ds(wid * BPW, BPW)])
        pltpu.sync_copy(vr_v, vx_out.at[pl.ds(wid * VPW, VPW)])

    return k(u_emb, v_emb, cidx2, vxidx2)


def _tc_score(u_rows, vx_rows):
    NBLK = 16
    BB = B // NBLK

    def body(u_ref, v0, n1, n2, n3, n4, n5, out_ref, a_pos, a_xs, a_m, a_e):
        i = pl.program_id(0)
        u = u_ref[...]
        s = jnp.sum(u * v0[...], axis=1)
        ls = jnp.minimum(s, 0.0) - jnp.log1p(jnp.exp(-jnp.abs(s)))
        negsum = n1[...] + n2[...] + n3[...] + n4[...] + n5[...]
        x = -jnp.sum(negsum * u, axis=1)
        bmax = jnp.max(x)
        bpos = jnp.full((1, 128), jnp.sum(ls), jnp.float32)
        bxs = jnp.full((1, 128), jnp.sum(x), jnp.float32)
        bm = jnp.full((1, 128), bmax, jnp.float32)
        be = jnp.full((1, 128), jnp.sum(jnp.exp(x - bmax)), jnp.float32)

        @pl.when(i == 0)
        def _():
            a_pos[...] = bpos
            a_xs[...] = bxs
            a_m[...] = bm
            a_e[...] = be

        @pl.when(i > 0)
        def _():
            m_old = a_m[...]
            m_new = jnp.maximum(m_old, bm)
            a_e[...] = a_e[...] * jnp.exp(m_old - m_new) + be * jnp.exp(bm - m_new)
            a_m[...] = m_new
            a_pos[...] = a_pos[...] + bpos
            a_xs[...] = a_xs[...] + bxs

        @pl.when(i == NBLK - 1)
        def _():
            lse = a_m[...] + jnp.log(a_e[...])
            loss_neg = a_xs[...] - jnp.float32(B) * lse
            out_ref[...] = -(a_pos[...] + loss_neg)

    out = pl.pallas_call(
        body,
        grid=(NBLK,),
        in_specs=[pl.BlockSpec((BB, DIM), lambda i: (i, 0))]
        + [pl.BlockSpec((BB, DIM), (lambda i, k=k: (k * NBLK + i, 0)))
           for k in range(NEG + 1)],
        out_specs=pl.BlockSpec((1, 128), lambda i: (0, 0)),
        out_shape=jax.ShapeDtypeStruct((1, 128), jnp.float32),
        scratch_shapes=[pltpu.VMEM((1, 128), jnp.float32) for _ in range(4)],
    )(u_rows, vx_rows, vx_rows, vx_rows, vx_rows, vx_rows, vx_rows)
    return out[0, 0]


def kernel(center, context, neg_v, u_emb, v_emb):
    center = center.astype(jnp.int32)
    context = context.astype(jnp.int32)
    neg_v = neg_v.astype(jnp.int32)
    vx_idx = jnp.concatenate([context, jnp.swapaxes(neg_v, 0, 1).reshape(-1)])
    cidx2 = center.reshape(B // CH, CH)
    vxidx2 = vx_idx.reshape((NEG + 1) * B // CH, CH)
    u_rows, vx_rows = _sc_gather(u_emb, v_emb, cidx2, vxidx2)
    return _tc_score(u_rows, vx_rows)
